# baseline (speedup 1.0000x reference)
import functools

import jax
import jax.numpy as jnp
from jax import lax
from jax.experimental import pallas as pl
from jax.experimental.pallas import tpu as pltpu
from jax.experimental.pallas import tpu_sc as plsc

S = 8192
D = 1024
DW = D // 2
B = 4096
SPAD = 8448

NC = 2
NS = 16
NW = NC * NS
SRPW = SPAD // NW
BRPW = B // NW
SCH = (40, 40, 40, 40, 40, 40, 24)
BCH = (40, 40, 40, 8)
ROWBUF = 40

BM = 256
WCH = 1056


def _pack_rows(src_f32, dst_i32, nrows):
    def body(r, _):
        for m in range(D // 32):
            a = src_f32[r, pl.ds(32 * m, 16)]
            b = src_f32[r, pl.ds(32 * m + 16, 16)]
            dst_i32[r, pl.ds(16 * m, 16)] = jnp.bitwise_or(
                lax.shift_right_logical(a, 16),
                jnp.bitwise_and(b, jnp.int32(-65536)))
        return 0

    lax.fori_loop(0, nrows, body, 0)


def _sc_body(table, ids_pad, labels, x, wpk, tpk, xpk,
             idx_s, idx_t, buf0, buf1, pk, sem0, sem1):
    wid = lax.axis_index("s") * NC + lax.axis_index("c")
    bufs = (buf0, buf1)
    sems = (sem0, sem1)

    def run_phase(chunks, mk_src, out, out_base):
        n = len(chunks)
        offs = [sum(chunks[:c]) for c in range(n)]
        cps = [None] * n
        cps[0] = pltpu.make_async_copy(
            mk_src(offs[0], chunks[0]), bufs[0].at[pl.ds(0, chunks[0])], sems[0])
        cps[0].start()
        for c in range(n):
            if c + 1 < n:
                cps[c + 1] = pltpu.make_async_copy(
                    mk_src(offs[c + 1], chunks[c + 1]),
                    bufs[(c + 1) % 2].at[pl.ds(0, chunks[c + 1])],
                    sems[(c + 1) % 2])
                cps[c + 1].start()
            cps[c].wait()
            _pack_rows(bufs[c % 2], pk, chunks[c])
            pltpu.sync_copy(pk.at[pl.ds(0, chunks[c])],
                            out.at[pl.ds(out_base + offs[c], chunks[c])])

    base_s = wid * SRPW
    pltpu.sync_copy(ids_pad.at[pl.ds(base_s, SRPW)], idx_s)
    run_phase(SCH, lambda off, sz: table.at[idx_s.at[pl.ds(off, sz)]],
              wpk, base_s)

    base_b = wid * BRPW
    pltpu.sync_copy(labels.at[pl.ds(base_b, BRPW)], idx_t)
    run_phase(BCH, lambda off, sz: table.at[idx_t.at[pl.ds(off, sz)]],
              tpk, base_b)

    run_phase(BCH, lambda off, sz: x.at[pl.ds(base_b + off, sz)],
              xpk, base_b)


@functools.cache
def _sc_gather():
    return pl.kernel(
        _sc_body,
        out_type=(
            jax.ShapeDtypeStruct((SPAD, DW), jnp.int32),
            jax.ShapeDtypeStruct((B, DW), jnp.int32),
            jax.ShapeDtypeStruct((B, DW), jnp.int32),
        ),
        mesh=plsc.VectorSubcoreMesh(core_axis_name="c", subcore_axis_name="s"),
        scratch_types=[
            pltpu.VMEM((SRPW,), jnp.int32),
            pltpu.VMEM((BRPW,), jnp.int32),
            pltpu.VMEM((ROWBUF, D), jnp.int32),
            pltpu.VMEM((ROWBUF, D), jnp.int32),
            pltpu.VMEM((ROWBUF, DW), jnp.int32),
            pltpu.SemaphoreType.DMA,
            pltpu.SemaphoreType.DMA,
        ],
    )


def _unpack_f32(v):
    lo = lax.bitcast_convert_type(lax.shift_left(v, 16), jnp.float32)
    hi = lax.bitcast_convert_type(lax.bitwise_and(v, jnp.int32(-65536)),
                                  jnp.float32)
    return lo, hi


def _main_body(xpk_ref, wpk_ref, tpk_ref, lab_ref, ids_ref, frq_ref, tf_ref,
               out_ref, wva_ref, wvb_ref, stage_ref, sem):
    i = pl.program_id(0)

    @pl.when(i == 0)
    def _():
        for c in range(SPAD // WCH):
            cp = pltpu.make_async_copy(
                wpk_ref.at[pl.ds(c * WCH, WCH)], stage_ref, sem)
            cp.start()
            cp.wait()
            lo, hi = _unpack_f32(stage_ref[...])
            wva_ref[pl.ds(c * WCH, WCH), :] = lo.astype(jnp.bfloat16)
            wvb_ref[pl.ds(c * WCH, WCH), :] = hi.astype(jnp.bfloat16)

    xlo, xhi = _unpack_f32(xpk_ref[...])
    tlo, thi = _unpack_f32(tpk_ref[...])
    tl = jnp.sum(xlo * tlo + xhi * thi, axis=1, keepdims=True) \
        - jnp.log(tf_ref[...])
    xa = xlo.astype(jnp.bfloat16)
    xb = xhi.astype(jnp.bfloat16)
    lab = lab_ref[...]
    dn = (((1,), (1,)), ((), ()))

    for n in range(S // 1024):
        acc = lax.dot_general(xa, wva_ref[pl.ds(1024 * n, 1024), :], dn,
                              preferred_element_type=jnp.float32)
        acc = acc + lax.dot_general(xb, wvb_ref[pl.ds(1024 * n, 1024), :], dn,
                                    preferred_element_type=jnp.float32)
        acc = acc - jnp.log(frq_ref[:, pl.ds(1024 * n, 1024)])
        acc = jnp.where(lab == ids_ref[:, pl.ds(1024 * n, 1024)],
                        jnp.float32(-1e37), acc)
        if n == 0:
            col = lax.broadcasted_iota(jnp.int32, acc.shape, 1)
            acc = jnp.where(col == 0, tl, acc)
        out_ref[:, pl.ds(1024 * n, 1024)] = acc

    acct = lax.dot_general(xa, wva_ref[pl.ds(S, 8), :], dn,
                           preferred_element_type=jnp.float32)
    acct = acct + lax.dot_general(xb, wvb_ref[pl.ds(S, 8), :], dn,
                                  preferred_element_type=jnp.float32)
    acct = acct - jnp.log(frq_ref[:, pl.ds(S, 8)])
    acct = jnp.where(lab == ids_ref[:, pl.ds(S, 8)], jnp.float32(-1e37), acct)
    out_ref[:, pl.ds(S, 1)] = acct[:, 0:1]


def _main(xpk, wpk, tpk, labels_col, ids_row, frq_row, tf_col):
    return pl.pallas_call(
        _main_body,
        grid=(B // BM,),
        in_specs=[
            pl.BlockSpec((BM, DW), lambda i: (i, 0)),
            pl.BlockSpec(memory_space=pl.ANY),
            pl.BlockSpec((BM, DW), lambda i: (i, 0)),
            pl.BlockSpec((BM, 1), lambda i: (i, 0)),
            pl.BlockSpec((1, SPAD), lambda i: (0, 0)),
            pl.BlockSpec((1, SPAD), lambda i: (0, 0)),
            pl.BlockSpec((BM, 1), lambda i: (i, 0)),
        ],
        out_specs=pl.BlockSpec((BM, S + 1), lambda i: (i, 0)),
        out_shape=jax.ShapeDtypeStruct((B, S + 1), jnp.float32),
        scratch_shapes=[
            pltpu.VMEM((SPAD, DW), jnp.bfloat16),
            pltpu.VMEM((SPAD, DW), jnp.bfloat16),
            pltpu.VMEM((WCH, DW), jnp.int32),
            pltpu.SemaphoreType.DMA,
        ],
        compiler_params=pltpu.CompilerParams(
            dimension_semantics=("arbitrary",),
        ),
    )(xpk, wpk, tpk, labels_col, ids_row, frq_row, tf_col)


def kernel(inputs, labels, weight, sample_ids, true_freq, sample_freq):
    labels_i = labels.astype(jnp.int32)
    ids_pad = jnp.concatenate([
        jnp.zeros((1,), jnp.int32),
        sample_ids.astype(jnp.int32),
        jnp.zeros((SPAD - S - 1,), jnp.int32),
    ])
    weight_i = lax.bitcast_convert_type(weight, jnp.int32)
    inputs_i = lax.bitcast_convert_type(inputs, jnp.int32)
    wpk, tpk, xpk = _sc_gather()(weight_i, ids_pad, labels_i, inputs_i)

    frq_row = jnp.concatenate([
        jnp.ones((1,), jnp.float32),
        sample_freq,
        jnp.ones((SPAD - S - 1,), jnp.float32),
    ]).reshape(1, SPAD)

    logits = _main(xpk, wpk, tpk, labels_i.reshape(B, 1),
                   ids_pad.reshape(1, SPAD), frq_row, true_freq.reshape(B, 1))
    return logits, jnp.zeros((B,), labels.dtype)


# speedup vs baseline: 2.0003x; 2.0003x over previous
import functools

import jax
import jax.numpy as jnp
from jax import lax
from jax.experimental import pallas as pl
from jax.experimental.pallas import tpu as pltpu
from jax.experimental.pallas import tpu_sc as plsc

S = 8192
D = 1024
B = 4096
SPAD = 8448
NROWS = SPAD + B

NC = 2
NS = 16
NW = NC * NS
RPW = NROWS // NW
CHUNK = 56
NCHUNK = RPW // CHUNK

BM = 256
WCH = 1056


def _sc_gather_body(table, ids, out, idx_v, rows0, rows1, sem0, sem1):
    wid = lax.axis_index("s") * NC + lax.axis_index("c")
    base = wid * RPW
    pltpu.sync_copy(ids.at[pl.ds(base, RPW)], idx_v)
    bufs = (rows0, rows1)
    sems = (sem0, sem1)
    cps = []
    for c in range(NCHUNK):
        cp = pltpu.make_async_copy(table.at[idx_v.at[pl.ds(c * CHUNK, CHUNK)]],
                                   bufs[c % 2], sems[c % 2])
        cp.start()
        cps.append(cp)
        if c > 0:
            cps[c - 1].wait()
            pltpu.sync_copy(bufs[(c - 1) % 2],
                            out.at[pl.ds(base + (c - 1) * CHUNK, CHUNK)])
    cps[NCHUNK - 1].wait()
    pltpu.sync_copy(bufs[(NCHUNK - 1) % 2],
                    out.at[pl.ds(base + (NCHUNK - 1) * CHUNK, CHUNK)])


@functools.cache
def _sc_gather():
    return pl.kernel(
        _sc_gather_body,
        out_type=jax.ShapeDtypeStruct((NROWS, D), jnp.float32),
        mesh=plsc.VectorSubcoreMesh(core_axis_name="c", subcore_axis_name="s"),
        scratch_types=[
            pltpu.VMEM((RPW,), jnp.int32),
            pltpu.VMEM((CHUNK, D), jnp.float32),
            pltpu.VMEM((CHUNK, D), jnp.float32),
            pltpu.SemaphoreType.DMA,
            pltpu.SemaphoreType.DMA,
        ],
    )


def _main_body(xbf_ref, whbm_ref, tw_ref, lab_ref, ids_ref, frq_ref, tf_ref,
               out_ref, wv_ref, stage_ref, sem):
    i = pl.program_id(0)

    @pl.when(i == 0)
    def _():
        for c in range(SPAD // WCH):
            cp = pltpu.make_async_copy(
                whbm_ref.at[pl.ds(c * WCH, WCH)], stage_ref, sem)
            cp.start()
            cp.wait()
            wv_ref[pl.ds(c * WCH, WCH), :] = stage_ref[...].astype(jnp.bfloat16)

    xb = xbf_ref[...]
    tl = jnp.sum(xb.astype(jnp.float32) * tw_ref[...],
                 axis=1, keepdims=True) - jnp.log(tf_ref[...])
    lab = lab_ref[...]

    for n in range(S // 1024):
        w = wv_ref[pl.ds(1024 * n, 1024), :]
        acc = lax.dot_general(xb, w, (((1,), (1,)), ((), ())),
                              preferred_element_type=jnp.float32)
        acc = acc - jnp.log(frq_ref[:, pl.ds(1024 * n, 1024)])
        acc = jnp.where(lab == ids_ref[:, pl.ds(1024 * n, 1024)],
                        jnp.float32(-1e37), acc)
        if n == 0:
            col = lax.broadcasted_iota(jnp.int32, acc.shape, 1)
            acc = jnp.where(col == 0, tl, acc)
        out_ref[:, pl.ds(1024 * n, 1024)] = acc

    wt = wv_ref[pl.ds(S, 8), :]
    acct = lax.dot_general(xb, wt, (((1,), (1,)), ((), ())),
                           preferred_element_type=jnp.float32)
    acct = acct - jnp.log(frq_ref[:, pl.ds(S, 8)])
    acct = jnp.where(lab == ids_ref[:, pl.ds(S, 8)], jnp.float32(-1e37), acct)
    out_ref[:, pl.ds(S, 1)] = acct[:, 0:1]


def _main(xbf, big, labels_col, ids_row, frq_row, tf_col):
    return pl.pallas_call(
        _main_body,
        grid=(B // BM,),
        in_specs=[
            pl.BlockSpec((BM, D), lambda i: (i, 0)),
            pl.BlockSpec(memory_space=pl.ANY),
            pl.BlockSpec((BM, D), lambda i: (i + SPAD // BM, 0)),
            pl.BlockSpec((BM, 1), lambda i: (i, 0)),
            pl.BlockSpec((1, SPAD), lambda i: (0, 0)),
            pl.BlockSpec((1, SPAD), lambda i: (0, 0)),
            pl.BlockSpec((BM, 1), lambda i: (i, 0)),
        ],
        out_specs=pl.BlockSpec((BM, S + 1), lambda i: (i, 0)),
        out_shape=jax.ShapeDtypeStruct((B, S + 1), jnp.float32),
        scratch_shapes=[
            pltpu.VMEM((SPAD, D), jnp.bfloat16),
            pltpu.VMEM((WCH, D), jnp.float32),
            pltpu.SemaphoreType.DMA,
        ],
        compiler_params=pltpu.CompilerParams(
            dimension_semantics=("arbitrary",),
        ),
    )(xbf, big, big, labels_col, ids_row, frq_row, tf_col)


def kernel(inputs, labels, weight, sample_ids, true_freq, sample_freq):
    labels_i = labels.astype(jnp.int32)
    ids_all = jnp.concatenate([
        jnp.zeros((1,), jnp.int32),
        sample_ids.astype(jnp.int32),
        jnp.zeros((SPAD - S - 1,), jnp.int32),
        labels_i,
    ])
    big = _sc_gather()(weight, ids_all)

    frq_row = jnp.concatenate([
        jnp.ones((1,), jnp.float32),
        sample_freq,
        jnp.ones((SPAD - S - 1,), jnp.float32),
    ]).reshape(1, SPAD)

    logits = _main(inputs.astype(jnp.bfloat16), big, labels_i.reshape(B, 1),
                   ids_all[:SPAD].reshape(1, SPAD), frq_row,
                   true_freq.reshape(B, 1))
    return logits, jnp.zeros((B,), labels.dtype)


# speedup vs baseline: 2.0420x; 1.0209x over previous
import functools

import jax
import jax.numpy as jnp
from jax import lax
from jax.experimental import pallas as pl
from jax.experimental.pallas import tpu as pltpu
from jax.experimental.pallas import tpu_sc as plsc

S = 8192
D = 1024
B = 4096
SPAD = 8448
NROWS = SPAD + B

NC = 2
NS = 16
NW = NC * NS
RPW = NROWS // NW
CHUNK = 56
NCHUNK = RPW // CHUNK

BM = 256
WCH = 1056


def _sc_gather_body(table, ids, out, idx_v, rows0, rows1, sem0, sem1):
    wid = lax.axis_index("s") * NC + lax.axis_index("c")
    base = wid * RPW
    pltpu.sync_copy(ids.at[pl.ds(base, RPW)], idx_v)
    bufs = (rows0, rows1)
    sems = (sem0, sem1)
    cps = []
    for c in range(NCHUNK):
        cp = pltpu.make_async_copy(table.at[idx_v.at[pl.ds(c * CHUNK, CHUNK)]],
                                   bufs[c % 2], sems[c % 2])
        cp.start()
        cps.append(cp)
        if c > 0:
            cps[c - 1].wait()
            pltpu.sync_copy(bufs[(c - 1) % 2],
                            out.at[pl.ds(base + (c - 1) * CHUNK, CHUNK)])
    cps[NCHUNK - 1].wait()
    pltpu.sync_copy(bufs[(NCHUNK - 1) % 2],
                    out.at[pl.ds(base + (NCHUNK - 1) * CHUNK, CHUNK)])


@functools.cache
def _sc_gather():
    return pl.kernel(
        _sc_gather_body,
        out_type=jax.ShapeDtypeStruct((NROWS, D), jnp.float32),
        mesh=plsc.VectorSubcoreMesh(core_axis_name="c", subcore_axis_name="s"),
        scratch_types=[
            pltpu.VMEM((RPW,), jnp.int32),
            pltpu.VMEM((CHUNK, D), jnp.float32),
            pltpu.VMEM((CHUNK, D), jnp.float32),
            pltpu.SemaphoreType.DMA,
            pltpu.SemaphoreType.DMA,
        ],
    )


def _main_body(xbf_ref, whbm_ref, tw_ref, lab_ref, ids_ref, frq_ref, tf_ref,
               out_ref, wv_ref, stage_ref, sem):
    i = pl.program_id(0)

    @pl.when(i == 0)
    def _():
        for c in range(SPAD // WCH):
            cp = pltpu.make_async_copy(
                whbm_ref.at[pl.ds(c * WCH, WCH)], stage_ref, sem)
            cp.start()
            cp.wait()
            wv_ref[pl.ds(c * WCH, WCH), :] = stage_ref[...].astype(jnp.bfloat16)

    xf = xbf_ref[...]
    xb = xf.astype(jnp.bfloat16)
    tl = jnp.sum(xf * tw_ref[...],
                 axis=1, keepdims=True) - jnp.log(tf_ref[...])
    lab = lab_ref[...]

    for n in range(S // 1024):
        w = wv_ref[pl.ds(1024 * n, 1024), :]
        acc = lax.dot_general(xb, w, (((1,), (1,)), ((), ())),
                              preferred_element_type=jnp.float32)
        acc = acc - jnp.log(frq_ref[:, pl.ds(1024 * n, 1024)])
        acc = jnp.where(lab == ids_ref[:, pl.ds(1024 * n, 1024)],
                        jnp.float32(-1e37), acc)
        if n == 0:
            col = lax.broadcasted_iota(jnp.int32, acc.shape, 1)
            acc = jnp.where(col == 0, tl, acc)
        out_ref[:, pl.ds(1024 * n, 1024)] = acc

    wt = wv_ref[pl.ds(S, 8), :]
    acct = lax.dot_general(xb, wt, (((1,), (1,)), ((), ())),
                           preferred_element_type=jnp.float32)
    acct = acct - jnp.log(frq_ref[:, pl.ds(S, 8)])
    acct = jnp.where(lab == ids_ref[:, pl.ds(S, 8)], jnp.float32(-1e37), acct)
    out_ref[:, pl.ds(S, 1)] = acct[:, 0:1]


def _main(xbf, big, labels_col, ids_row, frq_row, tf_col):
    return pl.pallas_call(
        _main_body,
        grid=(B // BM,),
        in_specs=[
            pl.BlockSpec((BM, D), lambda i: (i, 0)),
            pl.BlockSpec(memory_space=pl.ANY),
            pl.BlockSpec((BM, D), lambda i: (i + SPAD // BM, 0)),
            pl.BlockSpec((BM, 1), lambda i: (i, 0)),
            pl.BlockSpec((1, SPAD), lambda i: (0, 0)),
            pl.BlockSpec((1, SPAD), lambda i: (0, 0)),
            pl.BlockSpec((BM, 1), lambda i: (i, 0)),
        ],
        out_specs=pl.BlockSpec((BM, S + 1), lambda i: (i, 0)),
        out_shape=jax.ShapeDtypeStruct((B, S + 1), jnp.float32),
        scratch_shapes=[
            pltpu.VMEM((SPAD, D), jnp.bfloat16),
            pltpu.VMEM((WCH, D), jnp.float32),
            pltpu.SemaphoreType.DMA,
        ],
        compiler_params=pltpu.CompilerParams(
            dimension_semantics=("arbitrary",),
        ),
    )(xbf, big, big, labels_col, ids_row, frq_row, tf_col)


def kernel(inputs, labels, weight, sample_ids, true_freq, sample_freq):
    labels_i = labels.astype(jnp.int32)
    ids_all = jnp.concatenate([
        jnp.zeros((1,), jnp.int32),
        sample_ids.astype(jnp.int32),
        jnp.zeros((SPAD - S - 1,), jnp.int32),
        labels_i,
    ])
    big = _sc_gather()(weight, ids_all)

    frq_row = jnp.concatenate([
        jnp.ones((1,), jnp.float32),
        sample_freq,
        jnp.ones((SPAD - S - 1,), jnp.float32),
    ]).reshape(1, SPAD)

    logits = _main(inputs, big, labels_i.reshape(B, 1),
                   ids_all[:SPAD].reshape(1, SPAD), frq_row,
                   true_freq.reshape(B, 1))
    return logits, jnp.zeros((B,), labels.dtype)


# speedup vs baseline: 2.1037x; 1.0302x over previous
import functools

import jax
import jax.numpy as jnp
from jax import lax
from jax.experimental import pallas as pl
from jax.experimental.pallas import tpu as pltpu
from jax.experimental.pallas import tpu_sc as plsc

S = 8192
D = 1024
B = 4096
SPAD = 8448
NROWS = SPAD + B

NC = 2
NS = 16
NW = NC * NS
RPW = NROWS // NW
SCH = (40, 40, 40, 40, 40, 40, 40, 40, 40, 32)

BM = 256
WCH = 1056


def _sc_gather_body(table, ids, out, idx_v, rows0, rows1, rows2,
                    g0, g1, g2, w0, w1, w2):
    wid = lax.axis_index("s") * NC + lax.axis_index("c")
    base = wid * RPW
    pltpu.sync_copy(ids.at[pl.ds(base, RPW)], idx_v)
    bufs = (rows0, rows1, rows2)
    gsems = (g0, g1, g2)
    wsems = (w0, w1, w2)
    n = len(SCH)
    offs = [sum(SCH[:c]) for c in range(n)]
    g_cps = [None] * n
    wb_cps = [None] * n
    for c in range(n):
        if c >= 3:
            wb_cps[c - 3].wait()
        b = bufs[c % 3].at[pl.ds(0, SCH[c])]
        g_cps[c] = pltpu.make_async_copy(
            table.at[idx_v.at[pl.ds(offs[c], SCH[c])]], b, gsems[c % 3])
        g_cps[c].start()
        if c >= 1:
            g_cps[c - 1].wait()
            pb = bufs[(c - 1) % 3].at[pl.ds(0, SCH[c - 1])]
            wb_cps[c - 1] = pltpu.make_async_copy(
                pb, out.at[pl.ds(base + offs[c - 1], SCH[c - 1])],
                wsems[(c - 1) % 3])
            wb_cps[c - 1].start()
    g_cps[n - 1].wait()
    wb_cps[n - 1] = pltpu.make_async_copy(
        bufs[(n - 1) % 3].at[pl.ds(0, SCH[n - 1])],
        out.at[pl.ds(base + offs[n - 1], SCH[n - 1])], wsems[(n - 1) % 3])
    wb_cps[n - 1].start()
    for c in (n - 3, n - 2, n - 1):
        wb_cps[c].wait()


@functools.cache
def _sc_gather():
    return pl.kernel(
        _sc_gather_body,
        out_type=jax.ShapeDtypeStruct((NROWS, D), jnp.float32),
        mesh=plsc.VectorSubcoreMesh(core_axis_name="c", subcore_axis_name="s"),
        scratch_types=[
            pltpu.VMEM((RPW,), jnp.int32),
            pltpu.VMEM((40, D), jnp.float32),
            pltpu.VMEM((40, D), jnp.float32),
            pltpu.VMEM((40, D), jnp.float32),
            pltpu.SemaphoreType.DMA,
            pltpu.SemaphoreType.DMA,
            pltpu.SemaphoreType.DMA,
            pltpu.SemaphoreType.DMA,
            pltpu.SemaphoreType.DMA,
            pltpu.SemaphoreType.DMA,
        ],
    )


def _main_body(xbf_ref, whbm_ref, tw_ref, lab_ref, ids_ref, frq_ref, tf_ref,
               out_ref, wv_ref, stage_ref, stage2_ref, sem, sem2):
    i = pl.program_id(0)

    @pl.when(i == 0)
    def _():
        stages = (stage_ref, stage2_ref)
        sems = (sem, sem2)
        ncast = SPAD // WCH
        cps = [None] * ncast
        for c in range(ncast):
            cps[c] = pltpu.make_async_copy(
                whbm_ref.at[pl.ds(c * WCH, WCH)], stages[c % 2], sems[c % 2])
            cps[c].start()
            if c >= 1:
                cps[c - 1].wait()
                wv_ref[pl.ds((c - 1) * WCH, WCH), :] = \
                    stages[(c - 1) % 2][...].astype(jnp.bfloat16)
        cps[ncast - 1].wait()
        wv_ref[pl.ds((ncast - 1) * WCH, WCH), :] = \
            stages[(ncast - 1) % 2][...].astype(jnp.bfloat16)

    xf = xbf_ref[...]
    xb = xf.astype(jnp.bfloat16)
    tl = jnp.sum(xf * tw_ref[...],
                 axis=1, keepdims=True) - jnp.log(tf_ref[...])
    lab = lab_ref[...]

    for n in range(S // 1024):
        w = wv_ref[pl.ds(1024 * n, 1024), :]
        acc = lax.dot_general(xb, w, (((1,), (1,)), ((), ())),
                              preferred_element_type=jnp.float32)
        acc = acc - jnp.log(frq_ref[:, pl.ds(1024 * n, 1024)])
        acc = jnp.where(lab == ids_ref[:, pl.ds(1024 * n, 1024)],
                        jnp.float32(-1e37), acc)
        if n == 0:
            col = lax.broadcasted_iota(jnp.int32, acc.shape, 1)
            acc = jnp.where(col == 0, tl, acc)
        out_ref[:, pl.ds(1024 * n, 1024)] = acc

    wt = wv_ref[pl.ds(S, 8), :]
    acct = lax.dot_general(xb, wt, (((1,), (1,)), ((), ())),
                           preferred_element_type=jnp.float32)
    acct = acct - jnp.log(frq_ref[:, pl.ds(S, 8)])
    acct = jnp.where(lab == ids_ref[:, pl.ds(S, 8)], jnp.float32(-1e37), acct)
    out_ref[:, pl.ds(S, 1)] = acct[:, 0:1]


def _main(xbf, big, labels_col, ids_row, frq_row, tf_col):
    return pl.pallas_call(
        _main_body,
        grid=(B // BM,),
        in_specs=[
            pl.BlockSpec((BM, D), lambda i: (i, 0)),
            pl.BlockSpec(memory_space=pl.ANY),
            pl.BlockSpec((BM, D), lambda i: (i + SPAD // BM, 0)),
            pl.BlockSpec((BM, 1), lambda i: (i, 0)),
            pl.BlockSpec((1, SPAD), lambda i: (0, 0)),
            pl.BlockSpec((1, SPAD), lambda i: (0, 0)),
            pl.BlockSpec((BM, 1), lambda i: (i, 0)),
        ],
        out_specs=pl.BlockSpec((BM, S + 1), lambda i: (i, 0)),
        out_shape=jax.ShapeDtypeStruct((B, S + 1), jnp.float32),
        scratch_shapes=[
            pltpu.VMEM((SPAD, D), jnp.bfloat16),
            pltpu.VMEM((WCH, D), jnp.float32),
            pltpu.VMEM((WCH, D), jnp.float32),
            pltpu.SemaphoreType.DMA,
            pltpu.SemaphoreType.DMA,
        ],
        compiler_params=pltpu.CompilerParams(
            dimension_semantics=("arbitrary",),
        ),
    )(xbf, big, big, labels_col, ids_row, frq_row, tf_col)


def kernel(inputs, labels, weight, sample_ids, true_freq, sample_freq):
    labels_i = labels.astype(jnp.int32)
    ids_all = jnp.concatenate([
        jnp.zeros((1,), jnp.int32),
        sample_ids.astype(jnp.int32),
        jnp.zeros((SPAD - S - 1,), jnp.int32),
        labels_i,
    ])
    big = _sc_gather()(weight, ids_all)

    frq_row = jnp.concatenate([
        jnp.ones((1,), jnp.float32),
        sample_freq,
        jnp.ones((SPAD - S - 1,), jnp.float32),
    ]).reshape(1, SPAD)

    logits = _main(inputs, big, labels_i.reshape(B, 1),
                   ids_all[:SPAD].reshape(1, SPAD), frq_row,
                   true_freq.reshape(B, 1))
    return logits, jnp.zeros((B,), labels.dtype)
